# baseline (speedup 1.0000x reference)
import jax
import jax.numpy as jnp
from jax import lax
from jax.experimental import pallas as pl
from jax.experimental.pallas import tpu as pltpu
from jax.experimental.pallas import tpu_sc as plsc

N = 10000
E = 320000
D = 128
H = 128
DE = 16
INV_NORM = 1.0 / 100.0

_SC_CORES = 2
_SC_SUBCORES = 16
_GATHER_CHUNK = 400
_SCATTER_CHUNK = 160
_EDGE_BLOCK = 3200
_NODE_BLOCK = 1000
_NCHUNK = 5


def _silu(x):
    return x * (0.5 * jnp.tanh(x * 0.5) + 0.5)


def _ln(x, g, b, eps=1e-5):
    mu = jnp.mean(x, axis=-1, keepdims=True)
    ms = jnp.mean(x * x, axis=-1, keepdims=True)
    rstd = jax.lax.rsqrt(ms - mu * mu + eps)
    scale = rstd * g
    return x * scale + (b - mu * scale)


def _ln_silu(x, g, b, jdiv, eps=1e-5):
    mu = jnp.dot(x, jdiv, preferred_element_type=jnp.float32)
    ms = jnp.dot(x * x, jdiv, preferred_element_type=jnp.float32)
    rstd = jax.lax.rsqrt(ms - mu * mu + eps)
    scale = rstd * (0.5 * g)
    u = x * scale + (0.5 * b - mu * scale)
    return u * (jnp.tanh(u) + 1.0)


def _project(h, w1a, w1b, b1):
    nb = N // _NODE_BLOCK
    wspec = lambda shape: pl.BlockSpec(shape, lambda i: (0, 0))

    def body(hb, wa, wb, bb, o1, o2):
        o1[...] = jnp.dot(hb[...], wa[...],
                          preferred_element_type=jnp.float32) + bb[...]
        o2[...] = jnp.dot(hb[...], wb[...],
                          preferred_element_type=jnp.float32)

    return pl.pallas_call(
        body,
        grid=(nb,),
        in_specs=[
            pl.BlockSpec((_NODE_BLOCK, D), lambda i: (i, 0)),
            wspec((D, H)),
            wspec((D, H)),
            wspec((1, H)),
        ],
        out_specs=(pl.BlockSpec((_NODE_BLOCK, H), lambda i: (i, 0)),
                   pl.BlockSpec((_NODE_BLOCK, H), lambda i: (i, 0))),
        out_shape=(jax.ShapeDtypeStruct((N, H), jnp.float32),
                   jax.ShapeDtypeStruct((N, H), jnp.float32)),
        compiler_params=pltpu.CompilerParams(
            dimension_semantics=("parallel",)
        ),
    )(h, w1a, w1b, b1)


def _sc_gather_add(p1, p2, row3, col3):
    total = row3.shape[0] * _GATHER_CHUNK
    mesh = plsc.VectorSubcoreMesh(core_axis_name="c", subcore_axis_name="s")

    @pl.kernel(
        out_type=jax.ShapeDtypeStruct((total, H), jnp.float32),
        mesh=mesh,
        scratch_types=[],
    )
    def gk(r_hbm, c_hbm, p1_hbm, p2_hbm, out_hbm):
        def body(ir_vmem, ic_vmem, o_vmem):
            pltpu.sync_copy(p1_hbm.at[ir_vmem.at[0, 0]], o_vmem)
            pltpu.sync_copy(p2_hbm.at[ic_vmem.at[0, 0]], o_vmem, add=True)

        pltpu.emit_pipeline(
            body,
            grid=(total // _GATHER_CHUNK,),
            in_specs=[pl.BlockSpec((1, 1, _GATHER_CHUNK),
                                   index_map=lambda i: (i, 0, 0)),
                      pl.BlockSpec((1, 1, _GATHER_CHUNK),
                                   index_map=lambda i: (i, 0, 0))],
            out_specs=[pl.BlockSpec((_GATHER_CHUNK, H),
                                    index_map=lambda i: (i, 0))],
            core_axis_name=("c", "s"),
            dimension_semantics=(pltpu.PARALLEL,),
        )(r_hbm, c_hbm, out_hbm)

    return gk(row3, col3, p1, p2)


def _sc_scatter(mij, row3, zeros):
    mesh = plsc.VectorSubcoreMesh(core_axis_name="c", subcore_axis_name="s")
    nchunks = row3.shape[0]

    @pl.kernel(
        out_type=jax.ShapeDtypeStruct((_SC_CORES, N, H), jnp.float32),
        mesh=mesh,
        scratch_types=[
            pltpu.VMEM_SHARED((N, H), jnp.float32),
        ],
    )
    def sk(mij_hbm, idx_hbm, z_hbm, out_hbm, acc):
        cid = lax.axis_index("c")
        sid = lax.axis_index("s")

        @pl.when(sid == 0)
        def _():
            pltpu.sync_copy(z_hbm, acc)

        plsc.subcore_barrier()

        def body(idx_v, rows_v):
            pltpu.sync_copy(rows_v, acc.at[idx_v.at[0, 0]], add=True)

        pltpu.emit_pipeline(
            body,
            grid=(nchunks,),
            in_specs=[pl.BlockSpec((1, 1, _SCATTER_CHUNK),
                                   index_map=lambda i: (i, 0, 0)),
                      pl.BlockSpec((_SCATTER_CHUNK, H),
                                   index_map=lambda i: (i, 0))],
            out_specs=[],
            core_axis_name=("c", "s"),
            dimension_semantics=(pltpu.PARALLEL,),
        )(idx_hbm, mij_hbm)

        plsc.subcore_barrier()

        @pl.when(sid == 0)
        def _():
            pltpu.sync_copy(acc, out_hbm.at[cid])

    return sk(mij, row3, zeros)


def _sc_scatter_multi(mijs, row3s, zeros):
    n = len(mijs)
    mesh = plsc.VectorSubcoreMesh(core_axis_name="c", subcore_axis_name="s")
    grids = [r.shape[0] for r in row3s]

    @pl.kernel(
        out_type=jax.ShapeDtypeStruct((_SC_CORES, N, H), jnp.float32),
        mesh=mesh,
        scratch_types=[
            pltpu.VMEM_SHARED((N, H), jnp.float32),
        ],
    )
    def sk(*refs):
        mij_refs = refs[:n]
        idx_refs = refs[n:2 * n]
        z_hbm = refs[2 * n]
        out_hbm = refs[2 * n + 1]
        acc = refs[2 * n + 2]
        cid = lax.axis_index("c")
        sid = lax.axis_index("s")

        @pl.when(sid == 0)
        def _():
            pltpu.sync_copy(z_hbm, acc)

        plsc.subcore_barrier()

        def body(idx_v, rows_v):
            pltpu.sync_copy(rows_v, acc.at[idx_v.at[0, 0]], add=True)

        for j in range(n):
            pltpu.emit_pipeline(
                body,
                grid=(grids[j],),
                in_specs=[pl.BlockSpec((1, 1, _SCATTER_CHUNK),
                                       index_map=lambda i: (i, 0, 0)),
                          pl.BlockSpec((_SCATTER_CHUNK, H),
                                       index_map=lambda i: (i, 0))],
                out_specs=[],
                core_axis_name=("c", "s"),
                dimension_semantics=(pltpu.PARALLEL,),
            )(idx_refs[j], mij_refs[j])

        plsc.subcore_barrier()

        @pl.when(sid == 0)
        def _():
            pltpu.sync_copy(acc, out_hbm.at[cid])

    return sk(*mijs, *row3s, zeros)


def _edge_body(pre, ea_t, w1c, g1, be1, w2, b2, g2, be2, jdiv, out, outc):
    x = pre[...] + jax.lax.dot_general(
        ea_t[...], w1c[...], (((0,), (0,)), ((), ())),
        preferred_element_type=jnp.float32)
    x = _ln_silu(x, g1[...], be1[...], jdiv[...])
    y = jnp.dot(x, w2[...], preferred_element_type=jnp.float32) + b2[...]
    v = _ln_silu(y, g2[...], be2[...], jdiv[...])
    out[...] = v
    outc[...] = v


def _edge_mlp_chunk(pre, ea_t, chunk, mij_prev, w1c,
                    g1, be1, w2, b2, g2, be2, jdiv):
    ec = pre.shape[0]
    nb = ec // _EDGE_BLOCK
    base = chunk * nb
    wspec = lambda shape: pl.BlockSpec(shape, lambda i: (0, 0))
    in_specs = [
        pl.BlockSpec((_EDGE_BLOCK, H), lambda i: (i, 0)),
        pl.BlockSpec((DE, _EDGE_BLOCK), lambda i: (0, i + base)),
        wspec((DE, H)),
        wspec((1, H)),
        wspec((1, H)),
        wspec((H, H)),
        wspec((1, H)),
        wspec((1, H)),
        wspec((1, H)),
        wspec((H, H)),
    ]
    args = [pre, ea_t, w1c, g1, be1, w2, b2, g2, be2, jdiv]
    kwargs = {}
    body = _edge_body
    if mij_prev is not None:
        in_specs = [pl.BlockSpec(memory_space=pltpu.MemorySpace.HBM)] + in_specs
        args = [mij_prev] + args
        kwargs["input_output_aliases"] = {0: 0}
        body = lambda prev, *rest: _edge_body(*rest)
    return pl.pallas_call(
        body,
        grid=(nb,),
        in_specs=in_specs,
        out_specs=(pl.BlockSpec((_EDGE_BLOCK, H), lambda i: (i + base, 0)),
                   pl.BlockSpec((_EDGE_BLOCK, H), lambda i: (i, 0))),
        out_shape=(jax.ShapeDtypeStruct((E, H), jnp.float32),
                   jax.ShapeDtypeStruct((ec, H), jnp.float32)),
        compiler_params=pltpu.CompilerParams(
            dimension_semantics=("parallel",)
        ),
        **kwargs,
    )(*args)


def _node_body(hb, p0, p1, p2, p3, wn1a, wn1b, bn1, gn1, ben1, wn2, bn2,
               jdiv, out):
    agg = ((p0[0] + p1[0]) + (p2[0] + p3[0])) * INV_NORM
    x = jnp.dot(hb[...], wn1a[...], preferred_element_type=jnp.float32)
    x = x + jnp.dot(agg, wn1b[...], preferred_element_type=jnp.float32)
    x = x + bn1[...]
    x = _ln_silu(x, gn1[...], ben1[...], jdiv[...])
    out[...] = hb[...] + jnp.dot(x, wn2[...], preferred_element_type=jnp.float32) + bn2[...]


def _node_mlp(h, pA, pB, wn1a, wn1b, bn1, gn1, ben1, wn2, bn2, jdiv):
    nb = N // _NODE_BLOCK
    wspec = lambda shape: pl.BlockSpec(shape, lambda i: (0, 0))
    return pl.pallas_call(
        _node_body,
        grid=(nb,),
        in_specs=[
            pl.BlockSpec((_NODE_BLOCK, D), lambda i: (i, 0)),
            pl.BlockSpec((1, _NODE_BLOCK, H), lambda i: (0, i, 0)),
            pl.BlockSpec((1, _NODE_BLOCK, H), lambda i: (1, i, 0)),
            pl.BlockSpec((1, _NODE_BLOCK, H), lambda i: (0, i, 0)),
            pl.BlockSpec((1, _NODE_BLOCK, H), lambda i: (1, i, 0)),
            wspec((D, H)),
            wspec((H, H)),
            wspec((1, H)),
            wspec((1, H)),
            wspec((1, H)),
            wspec((H, D)),
            wspec((1, D)),
            wspec((H, H)),
        ],
        out_specs=pl.BlockSpec((_NODE_BLOCK, D), lambda i: (i, 0)),
        out_shape=jax.ShapeDtypeStruct((N, D), jnp.float32),
        compiler_params=pltpu.CompilerParams(
            dimension_semantics=("parallel",)
        ),
    )(h, pA, pA, pB, pB, wn1a, wn1b, bn1, gn1, ben1, wn2, bn2, jdiv)


def kernel(h, edge_index, edge_attr, W1, b1, g1, be1, W2, b2, g2, be2,
           Wn1, bn1, gn1, ben1, Wn2, bn2):
    ec = E // _NCHUNK
    row = edge_index[0].astype(jnp.int32)
    col = edge_index[1].astype(jnp.int32)

    w1a, w1b, w1c = W1[:D], W1[D:2 * D], W1[2 * D:]
    r = lambda v: v.reshape(1, -1)

    p1, p2 = _project(h, w1a, w1b, r(b1))
    ea_t = edge_attr.T
    jdiv = jnp.full((H, H), 1.0 / H, jnp.float32)

    zeros = jnp.zeros((N, H), jnp.float32)
    mij = None
    chunk_mijs = []
    chunk_rows = []
    for k in range(_NCHUNK):
        row3k = row[k * ec:(k + 1) * ec].reshape(
            ec // _GATHER_CHUNK, 1, _GATHER_CHUNK)
        col3k = col[k * ec:(k + 1) * ec].reshape(
            ec // _GATHER_CHUNK, 1, _GATHER_CHUNK)
        pre = _sc_gather_add(p1, p2, row3k, col3k)
        mij, mij_k = _edge_mlp_chunk(pre, ea_t, k, mij, w1c,
                                     r(g1), r(be1), W2, r(b2), r(g2), r(be2),
                                     jdiv)
        chunk_mijs.append(mij_k)
        chunk_rows.append(row[k * ec:(k + 1) * ec].reshape(
            ec // _SCATTER_CHUNK, 1, _SCATTER_CHUNK))

    pA = _sc_scatter_multi(chunk_mijs[:3], chunk_rows[:3], zeros)
    pB = _sc_scatter_multi(chunk_mijs[3:], chunk_rows[3:], zeros)

    h_out = _node_mlp(h, pA, pB, Wn1[:D], Wn1[D:],
                      r(bn1), r(gn1), r(ben1), Wn2, r(bn2), jdiv)
    return (h_out, mij)


# speedup vs baseline: 1.0530x; 1.0530x over previous
import jax
import jax.numpy as jnp
from jax import lax
from jax.experimental import pallas as pl
from jax.experimental.pallas import tpu as pltpu
from jax.experimental.pallas import tpu_sc as plsc

N = 10000
E = 320000
D = 128
H = 128
DE = 16
INV_NORM = 1.0 / 100.0

_SC_CORES = 2
_SC_SUBCORES = 16
_GATHER_CHUNK = 400
_SCATTER_CHUNK = 160
_EDGE_BLOCK = 6400
_NODE_BLOCK = 1000
_NCHUNK = 5


def _silu(x):
    return x * (0.5 * jnp.tanh(x * 0.5) + 0.5)


def _ln(x, g, b, eps=1e-5):
    mu = jnp.mean(x, axis=-1, keepdims=True)
    ms = jnp.mean(x * x, axis=-1, keepdims=True)
    rstd = jax.lax.rsqrt(ms - mu * mu + eps)
    scale = rstd * g
    return x * scale + (b - mu * scale)


def _ln_silu(x, g, b, jdiv, eps=1e-5):
    mu = jnp.dot(x, jdiv, preferred_element_type=jnp.float32)
    ms = jnp.dot(x * x, jdiv, preferred_element_type=jnp.float32)
    rstd = jax.lax.rsqrt(ms - mu * mu + eps)
    scale = rstd * (0.5 * g)
    u = x * scale + (0.5 * b - mu * scale)
    return u * (jnp.tanh(u) + 1.0)


def _project(h, w1a, w1b, b1):
    nb = N // _NODE_BLOCK
    wspec = lambda shape: pl.BlockSpec(shape, lambda i: (0, 0))

    def body(hb, wa, wb, bb, o1, o2):
        o1[...] = jnp.dot(hb[...], wa[...],
                          preferred_element_type=jnp.float32) + bb[...]
        o2[...] = jnp.dot(hb[...], wb[...],
                          preferred_element_type=jnp.float32)

    return pl.pallas_call(
        body,
        grid=(nb,),
        in_specs=[
            pl.BlockSpec((_NODE_BLOCK, D), lambda i: (i, 0)),
            wspec((D, H)),
            wspec((D, H)),
            wspec((1, H)),
        ],
        out_specs=(pl.BlockSpec((_NODE_BLOCK, H), lambda i: (i, 0)),
                   pl.BlockSpec((_NODE_BLOCK, H), lambda i: (i, 0))),
        out_shape=(jax.ShapeDtypeStruct((N, H), jnp.float32),
                   jax.ShapeDtypeStruct((N, H), jnp.float32)),
        compiler_params=pltpu.CompilerParams(
            dimension_semantics=("parallel",)
        ),
    )(h, w1a, w1b, b1)


def _sc_gather_add(p1, p2, row3, col3):
    total = row3.shape[0] * _GATHER_CHUNK
    mesh = plsc.VectorSubcoreMesh(core_axis_name="c", subcore_axis_name="s")

    @pl.kernel(
        out_type=jax.ShapeDtypeStruct((total, H), jnp.float32),
        mesh=mesh,
        scratch_types=[],
    )
    def gk(r_hbm, c_hbm, p1_hbm, p2_hbm, out_hbm):
        def body(ir_vmem, ic_vmem, o_vmem):
            pltpu.sync_copy(p1_hbm.at[ir_vmem.at[0, 0]], o_vmem)
            pltpu.sync_copy(p2_hbm.at[ic_vmem.at[0, 0]], o_vmem, add=True)

        pltpu.emit_pipeline(
            body,
            grid=(total // _GATHER_CHUNK,),
            in_specs=[pl.BlockSpec((1, 1, _GATHER_CHUNK),
                                   index_map=lambda i: (i, 0, 0)),
                      pl.BlockSpec((1, 1, _GATHER_CHUNK),
                                   index_map=lambda i: (i, 0, 0))],
            out_specs=[pl.BlockSpec((_GATHER_CHUNK, H),
                                    index_map=lambda i: (i, 0))],
            core_axis_name=("c", "s"),
            dimension_semantics=(pltpu.PARALLEL,),
        )(r_hbm, c_hbm, out_hbm)

    return gk(row3, col3, p1, p2)


def _sc_scatter(mij, row3, zeros):
    mesh = plsc.VectorSubcoreMesh(core_axis_name="c", subcore_axis_name="s")
    nchunks = row3.shape[0]

    @pl.kernel(
        out_type=jax.ShapeDtypeStruct((_SC_CORES, N, H), jnp.float32),
        mesh=mesh,
        scratch_types=[
            pltpu.VMEM_SHARED((N, H), jnp.float32),
        ],
    )
    def sk(mij_hbm, idx_hbm, z_hbm, out_hbm, acc):
        cid = lax.axis_index("c")
        sid = lax.axis_index("s")

        @pl.when(sid == 0)
        def _():
            pltpu.sync_copy(z_hbm, acc)

        plsc.subcore_barrier()

        def body(idx_v, rows_v):
            pltpu.sync_copy(rows_v, acc.at[idx_v.at[0, 0]], add=True)

        pltpu.emit_pipeline(
            body,
            grid=(nchunks,),
            in_specs=[pl.BlockSpec((1, 1, _SCATTER_CHUNK),
                                   index_map=lambda i: (i, 0, 0)),
                      pl.BlockSpec((_SCATTER_CHUNK, H),
                                   index_map=lambda i: (i, 0))],
            out_specs=[],
            core_axis_name=("c", "s"),
            dimension_semantics=(pltpu.PARALLEL,),
        )(idx_hbm, mij_hbm)

        plsc.subcore_barrier()

        @pl.when(sid == 0)
        def _():
            pltpu.sync_copy(acc, out_hbm.at[cid])

    return sk(mij, row3, zeros)


def _edge_body(pre, ea_t, w1c, g1, be1, w2, b2, g2, be2, jdiv, out):
    x = pre[...] + jax.lax.dot_general(
        ea_t[...], w1c[...], (((0,), (0,)), ((), ())),
        preferred_element_type=jnp.float32)
    x = _ln_silu(x, g1[...], be1[...], jdiv[...])
    y = jnp.dot(x, w2[...], preferred_element_type=jnp.float32) + b2[...]
    out[...] = _ln_silu(y, g2[...], be2[...], jdiv[...])


def _edge_mlp_chunk(pre, ea_t, chunk, mij_prev, w1c,
                    g1, be1, w2, b2, g2, be2, jdiv):
    ec = pre.shape[0]
    nb = ec // _EDGE_BLOCK
    base = chunk * nb
    wspec = lambda shape: pl.BlockSpec(shape, lambda i: (0, 0))
    in_specs = [
        pl.BlockSpec((_EDGE_BLOCK, H), lambda i: (i, 0)),
        pl.BlockSpec((DE, _EDGE_BLOCK), lambda i: (0, i + base)),
        wspec((DE, H)),
        wspec((1, H)),
        wspec((1, H)),
        wspec((H, H)),
        wspec((1, H)),
        wspec((1, H)),
        wspec((1, H)),
        wspec((H, H)),
    ]
    args = [pre, ea_t, w1c, g1, be1, w2, b2, g2, be2, jdiv]
    kwargs = {}
    body = _edge_body
    if mij_prev is not None:
        in_specs = [pl.BlockSpec(memory_space=pltpu.MemorySpace.HBM)] + in_specs
        args = [mij_prev] + args
        kwargs["input_output_aliases"] = {0: 0}
        body = lambda prev, *rest: _edge_body(*rest)
    return pl.pallas_call(
        body,
        grid=(nb,),
        in_specs=in_specs,
        out_specs=pl.BlockSpec((_EDGE_BLOCK, H), lambda i: (i + base, 0)),
        out_shape=jax.ShapeDtypeStruct((E, H), jnp.float32),
        compiler_params=pltpu.CompilerParams(
            dimension_semantics=("parallel",)
        ),
        **kwargs,
    )(*args)


def _node_body(hb, p0, p1, wn1a, wn1b, bn1, gn1, ben1, wn2, bn2, jdiv, out):
    agg = (p0[0] + p1[0]) * INV_NORM
    x = jnp.dot(hb[...], wn1a[...], preferred_element_type=jnp.float32)
    x = x + jnp.dot(agg, wn1b[...], preferred_element_type=jnp.float32)
    x = x + bn1[...]
    x = _ln_silu(x, gn1[...], ben1[...], jdiv[...])
    out[...] = hb[...] + jnp.dot(x, wn2[...], preferred_element_type=jnp.float32) + bn2[...]


def _node_mlp(h, partials, wn1a, wn1b, bn1, gn1, ben1, wn2, bn2, jdiv):
    nb = N // _NODE_BLOCK
    wspec = lambda shape: pl.BlockSpec(shape, lambda i: (0, 0))
    return pl.pallas_call(
        _node_body,
        grid=(nb,),
        in_specs=[
            pl.BlockSpec((_NODE_BLOCK, D), lambda i: (i, 0)),
            pl.BlockSpec((1, _NODE_BLOCK, H), lambda i: (0, i, 0)),
            pl.BlockSpec((1, _NODE_BLOCK, H), lambda i: (1, i, 0)),
            wspec((D, H)),
            wspec((H, H)),
            wspec((1, H)),
            wspec((1, H)),
            wspec((1, H)),
            wspec((H, D)),
            wspec((1, D)),
            wspec((H, H)),
        ],
        out_specs=pl.BlockSpec((_NODE_BLOCK, D), lambda i: (i, 0)),
        out_shape=jax.ShapeDtypeStruct((N, D), jnp.float32),
        compiler_params=pltpu.CompilerParams(
            dimension_semantics=("parallel",)
        ),
    )(h, partials, partials, wn1a, wn1b, bn1, gn1, ben1, wn2, bn2, jdiv)


def kernel(h, edge_index, edge_attr, W1, b1, g1, be1, W2, b2, g2, be2,
           Wn1, bn1, gn1, ben1, Wn2, bn2):
    ec = E // _NCHUNK
    row = edge_index[0].astype(jnp.int32)
    col = edge_index[1].astype(jnp.int32)
    srow3 = row.reshape(E // _SCATTER_CHUNK, 1, _SCATTER_CHUNK)

    w1a, w1b, w1c = W1[:D], W1[D:2 * D], W1[2 * D:]
    r = lambda v: v.reshape(1, -1)

    p1, p2 = _project(h, w1a, w1b, r(b1))
    ea_t = edge_attr.T
    jdiv = jnp.full((H, H), 1.0 / H, jnp.float32)

    mij = None
    for k in range(_NCHUNK):
        row3k = row[k * ec:(k + 1) * ec].reshape(
            ec // _GATHER_CHUNK, 1, _GATHER_CHUNK)
        col3k = col[k * ec:(k + 1) * ec].reshape(
            ec // _GATHER_CHUNK, 1, _GATHER_CHUNK)
        pre = _sc_gather_add(p1, p2, row3k, col3k)
        mij = _edge_mlp_chunk(pre, ea_t, k, mij, w1c,
                              r(g1), r(be1), W2, r(b2), r(g2), r(be2), jdiv)

    zeros = jnp.zeros((N, H), jnp.float32)
    partials = _sc_scatter(mij, srow3, zeros)

    h_out = _node_mlp(h, partials, Wn1[:D], Wn1[D:],
                      r(bn1), r(gn1), r(ben1), Wn2, r(bn2), jdiv)
    return (h_out, mij)


# speedup vs baseline: 1.0536x; 1.0006x over previous
import jax
import jax.numpy as jnp
from jax import lax
from jax.experimental import pallas as pl
from jax.experimental.pallas import tpu as pltpu
from jax.experimental.pallas import tpu_sc as plsc

N = 10000
E = 320000
D = 128
H = 128
DE = 16
INV_NORM = 1.0 / 100.0

_SC_CORES = 2
_SC_SUBCORES = 16
_GATHER_CHUNK = 400
_SCATTER_CHUNK = 160
_EDGE_BLOCK = 12800
_NODE_BLOCK = 1000
_NCHUNK = 5


def _silu(x):
    return x * (0.5 * jnp.tanh(x * 0.5) + 0.5)


def _ln(x, g, b, eps=1e-5):
    mu = jnp.mean(x, axis=-1, keepdims=True)
    ms = jnp.mean(x * x, axis=-1, keepdims=True)
    rstd = jax.lax.rsqrt(ms - mu * mu + eps)
    scale = rstd * g
    return x * scale + (b - mu * scale)


def _ln_silu(x, g, b, jdiv, eps=1e-5):
    mu = jnp.dot(x, jdiv, preferred_element_type=jnp.float32)
    ms = jnp.dot(x * x, jdiv, preferred_element_type=jnp.float32)
    rstd = jax.lax.rsqrt(ms - mu * mu + eps)
    scale = rstd * (0.5 * g)
    u = x * scale + (0.5 * b - mu * scale)
    return u * (jnp.tanh(u) + 1.0)


def _project(h, w1a, w1b, b1):
    nb = N // _NODE_BLOCK
    wspec = lambda shape: pl.BlockSpec(shape, lambda i: (0, 0))

    def body(hb, wa, wb, bb, o1, o2):
        o1[...] = jnp.dot(hb[...], wa[...],
                          preferred_element_type=jnp.float32) + bb[...]
        o2[...] = jnp.dot(hb[...], wb[...],
                          preferred_element_type=jnp.float32)

    return pl.pallas_call(
        body,
        grid=(nb,),
        in_specs=[
            pl.BlockSpec((_NODE_BLOCK, D), lambda i: (i, 0)),
            wspec((D, H)),
            wspec((D, H)),
            wspec((1, H)),
        ],
        out_specs=(pl.BlockSpec((_NODE_BLOCK, H), lambda i: (i, 0)),
                   pl.BlockSpec((_NODE_BLOCK, H), lambda i: (i, 0))),
        out_shape=(jax.ShapeDtypeStruct((N, H), jnp.float32),
                   jax.ShapeDtypeStruct((N, H), jnp.float32)),
        compiler_params=pltpu.CompilerParams(
            dimension_semantics=("parallel",)
        ),
    )(h, w1a, w1b, b1)


def _sc_gather_add(p1, p2, row3, col3):
    total = row3.shape[0] * _GATHER_CHUNK
    mesh = plsc.VectorSubcoreMesh(core_axis_name="c", subcore_axis_name="s")

    @pl.kernel(
        out_type=jax.ShapeDtypeStruct((total, H), jnp.float32),
        mesh=mesh,
        scratch_types=[],
    )
    def gk(r_hbm, c_hbm, p1_hbm, p2_hbm, out_hbm):
        def body(ir_vmem, ic_vmem, o_vmem):
            pltpu.sync_copy(p1_hbm.at[ir_vmem.at[0, 0]], o_vmem)
            pltpu.sync_copy(p2_hbm.at[ic_vmem.at[0, 0]], o_vmem, add=True)

        pltpu.emit_pipeline(
            body,
            grid=(total // _GATHER_CHUNK,),
            in_specs=[pl.BlockSpec((1, 1, _GATHER_CHUNK),
                                   index_map=lambda i: (i, 0, 0)),
                      pl.BlockSpec((1, 1, _GATHER_CHUNK),
                                   index_map=lambda i: (i, 0, 0))],
            out_specs=[pl.BlockSpec((_GATHER_CHUNK, H),
                                    index_map=lambda i: (i, 0))],
            core_axis_name=("c", "s"),
            dimension_semantics=(pltpu.PARALLEL,),
        )(r_hbm, c_hbm, out_hbm)

    return gk(row3, col3, p1, p2)


def _sc_scatter(mij, row3, zeros):
    mesh = plsc.VectorSubcoreMesh(core_axis_name="c", subcore_axis_name="s")
    nchunks = row3.shape[0]

    @pl.kernel(
        out_type=jax.ShapeDtypeStruct((_SC_CORES, N, H), jnp.float32),
        mesh=mesh,
        scratch_types=[
            pltpu.VMEM_SHARED((N, H), jnp.float32),
        ],
    )
    def sk(mij_hbm, idx_hbm, z_hbm, out_hbm, acc):
        cid = lax.axis_index("c")
        sid = lax.axis_index("s")

        @pl.when(sid == 0)
        def _():
            pltpu.sync_copy(z_hbm, acc)

        plsc.subcore_barrier()

        def body(idx_v, rows_v):
            pltpu.sync_copy(rows_v, acc.at[idx_v.at[0, 0]], add=True)

        pltpu.emit_pipeline(
            body,
            grid=(nchunks,),
            in_specs=[pl.BlockSpec((1, 1, _SCATTER_CHUNK),
                                   index_map=lambda i: (i, 0, 0)),
                      pl.BlockSpec((_SCATTER_CHUNK, H),
                                   index_map=lambda i: (i, 0))],
            out_specs=[],
            core_axis_name=("c", "s"),
            dimension_semantics=(pltpu.PARALLEL,),
        )(idx_hbm, mij_hbm)

        plsc.subcore_barrier()

        @pl.when(sid == 0)
        def _():
            pltpu.sync_copy(acc, out_hbm.at[cid])

    return sk(mij, row3, zeros)


def _edge_body(pre, ea_t, w1c, g1, be1, w2, b2, g2, be2, jdiv, out):
    x = pre[...] + jax.lax.dot_general(
        ea_t[...], w1c[...], (((0,), (0,)), ((), ())),
        preferred_element_type=jnp.float32)
    x = _ln_silu(x, g1[...], be1[...], jdiv[...])
    y = jnp.dot(x, w2[...], preferred_element_type=jnp.float32) + b2[...]
    out[...] = _ln_silu(y, g2[...], be2[...], jdiv[...])


def _edge_mlp_chunk(pre, ea_t, chunk, mij_prev, w1c,
                    g1, be1, w2, b2, g2, be2, jdiv):
    ec = pre.shape[0]
    nb = ec // _EDGE_BLOCK
    base = chunk * nb
    wspec = lambda shape: pl.BlockSpec(shape, lambda i: (0, 0))
    in_specs = [
        pl.BlockSpec((_EDGE_BLOCK, H), lambda i: (i, 0)),
        pl.BlockSpec((DE, _EDGE_BLOCK), lambda i: (0, i + base)),
        wspec((DE, H)),
        wspec((1, H)),
        wspec((1, H)),
        wspec((H, H)),
        wspec((1, H)),
        wspec((1, H)),
        wspec((1, H)),
        wspec((H, H)),
    ]
    args = [pre, ea_t, w1c, g1, be1, w2, b2, g2, be2, jdiv]
    kwargs = {}
    body = _edge_body
    if mij_prev is not None:
        in_specs = [pl.BlockSpec(memory_space=pltpu.MemorySpace.HBM)] + in_specs
        args = [mij_prev] + args
        kwargs["input_output_aliases"] = {0: 0}
        body = lambda prev, *rest: _edge_body(*rest)
    return pl.pallas_call(
        body,
        grid=(nb,),
        in_specs=in_specs,
        out_specs=pl.BlockSpec((_EDGE_BLOCK, H), lambda i: (i + base, 0)),
        out_shape=jax.ShapeDtypeStruct((E, H), jnp.float32),
        compiler_params=pltpu.CompilerParams(
            dimension_semantics=("parallel",)
        ),
        **kwargs,
    )(*args)


def _node_body(hb, p0, p1, wn1a, wn1b, bn1, gn1, ben1, wn2, bn2, jdiv, out):
    agg = (p0[0] + p1[0]) * INV_NORM
    x = jnp.dot(hb[...], wn1a[...], preferred_element_type=jnp.float32)
    x = x + jnp.dot(agg, wn1b[...], preferred_element_type=jnp.float32)
    x = x + bn1[...]
    x = _ln_silu(x, gn1[...], ben1[...], jdiv[...])
    out[...] = hb[...] + jnp.dot(x, wn2[...], preferred_element_type=jnp.float32) + bn2[...]


def _node_mlp(h, partials, wn1a, wn1b, bn1, gn1, ben1, wn2, bn2, jdiv):
    nb = N // _NODE_BLOCK
    wspec = lambda shape: pl.BlockSpec(shape, lambda i: (0, 0))
    return pl.pallas_call(
        _node_body,
        grid=(nb,),
        in_specs=[
            pl.BlockSpec((_NODE_BLOCK, D), lambda i: (i, 0)),
            pl.BlockSpec((1, _NODE_BLOCK, H), lambda i: (0, i, 0)),
            pl.BlockSpec((1, _NODE_BLOCK, H), lambda i: (1, i, 0)),
            wspec((D, H)),
            wspec((H, H)),
            wspec((1, H)),
            wspec((1, H)),
            wspec((1, H)),
            wspec((H, D)),
            wspec((1, D)),
            wspec((H, H)),
        ],
        out_specs=pl.BlockSpec((_NODE_BLOCK, D), lambda i: (i, 0)),
        out_shape=jax.ShapeDtypeStruct((N, D), jnp.float32),
        compiler_params=pltpu.CompilerParams(
            dimension_semantics=("parallel",)
        ),
    )(h, partials, partials, wn1a, wn1b, bn1, gn1, ben1, wn2, bn2, jdiv)


def kernel(h, edge_index, edge_attr, W1, b1, g1, be1, W2, b2, g2, be2,
           Wn1, bn1, gn1, ben1, Wn2, bn2):
    ec = E // _NCHUNK
    row = edge_index[0].astype(jnp.int32)
    col = edge_index[1].astype(jnp.int32)
    srow3 = row.reshape(E // _SCATTER_CHUNK, 1, _SCATTER_CHUNK)

    w1a, w1b, w1c = W1[:D], W1[D:2 * D], W1[2 * D:]
    r = lambda v: v.reshape(1, -1)

    p1, p2 = _project(h, w1a, w1b, r(b1))
    ea_t = edge_attr.T
    jdiv = jnp.full((H, H), 1.0 / H, jnp.float32)

    mij = None
    for k in range(_NCHUNK):
        row3k = row[k * ec:(k + 1) * ec].reshape(
            ec // _GATHER_CHUNK, 1, _GATHER_CHUNK)
        col3k = col[k * ec:(k + 1) * ec].reshape(
            ec // _GATHER_CHUNK, 1, _GATHER_CHUNK)
        pre = _sc_gather_add(p1, p2, row3k, col3k)
        mij = _edge_mlp_chunk(pre, ea_t, k, mij, w1c,
                              r(g1), r(be1), W2, r(b2), r(g2), r(be2), jdiv)

    zeros = jnp.zeros((N, H), jnp.float32)
    partials = _sc_scatter(mij, srow3, zeros)

    h_out = _node_mlp(h, partials, Wn1[:D], Wn1[D:],
                      r(bn1), r(gn1), r(ben1), Wn2, r(bn2), jdiv)
    return (h_out, mij)
